# baseline (speedup 1.0000x reference)
import jax
import jax.numpy as jnp
from jax.experimental import pallas as pl
from jax.experimental.pallas import tpu as pltpu

_N_ITERS = 20
_STRIP = 512


_UNROLL = 1


def _normalize_in_place(e_ref, n):
    ns = n // _STRIP

    def sweep0(s, c_acc):
        rows = pl.ds(s * _STRIP, _STRIP)
        z = e_ref[rows, :]
        m = jnp.max(z, axis=1, keepdims=True)
        e = jnp.exp(z - m)
        e_ref[rows, :] = e
        r = jnp.sum(e, axis=1, keepdims=True)
        a = 1.0 / r
        ae = e_ref[rows, :] * a
        return c_acc + jnp.sum(ae, axis=0, keepdims=True)

    c = jax.lax.fori_loop(0, ns, sweep0, jnp.zeros((1, n), jnp.float32))
    b = 1.0 / c

    ones_col = jnp.ones((n, 128), jnp.float32)

    def iter_body(_, b):
        def sweep(s, c_acc):
            rows = pl.ds(s * _STRIP, _STRIP)
            f = e_ref[rows, :] * b
            r = jnp.dot(f, ones_col, preferred_element_type=jnp.float32)
            a = 1.0 / r[:, 0:1]
            ae = e_ref[rows, :] * a
            return c_acc + jnp.sum(ae, axis=0, keepdims=True)

        c = jax.lax.fori_loop(0, ns, sweep, jnp.zeros((1, n), jnp.float32))
        return 1.0 / c

    b_prev = jax.lax.fori_loop(0, _N_ITERS - 2, iter_body, b)
    b_last = iter_body(0, b_prev)

    w = b_last / b_prev

    def final_sweep(s, carry):
        rows = pl.ds(s * _STRIP, _STRIP)
        f = e_ref[rows, :] * b_prev
        r = jnp.sum(f, axis=1, keepdims=True)
        a = 1.0 / r
        e_ref[rows, :] = (f * a) * w
        return carry

    jax.lax.fori_loop(0, ns, final_sweep, 0)


def _sinkhorn_body(x_hbm, o_hbm, e_scr, in_sems, out_sems):
    i = pl.program_id(0)
    nb = pl.num_programs(0)
    n = e_scr.shape[1]
    buf = jax.lax.rem(i, 2)
    other = 1 - buf

    @pl.when(i == 0)
    def _():
        pltpu.make_async_copy(x_hbm.at[0], e_scr.at[0], in_sems.at[0]).start()
        pltpu.make_async_copy(x_hbm.at[1], e_scr.at[1], in_sems.at[1]).start()

    pltpu.make_async_copy(x_hbm.at[i], e_scr.at[buf], in_sems.at[buf]).wait()

    _normalize_in_place(e_scr.at[buf], n)

    pltpu.make_async_copy(e_scr.at[buf], o_hbm.at[i], out_sems.at[buf]).start()

    @pl.when(jnp.logical_and(i >= 1, i < nb - 1))
    def _():
        pltpu.make_async_copy(e_scr.at[other], o_hbm.at[i - 1],
                              out_sems.at[other]).wait()
        pltpu.make_async_copy(x_hbm.at[i + 1], e_scr.at[other],
                              in_sems.at[other]).start()

    @pl.when(i == nb - 1)
    def _():
        pltpu.make_async_copy(e_scr.at[other], o_hbm.at[i - 1],
                              out_sems.at[other]).wait()
        pltpu.make_async_copy(e_scr.at[buf], o_hbm.at[i],
                              out_sems.at[buf]).wait()


def kernel(log_alpha):
    batch, n, _ = log_alpha.shape
    return pl.pallas_call(
        _sinkhorn_body,
        out_shape=jax.ShapeDtypeStruct((batch, n, n), jnp.float32),
        grid=(batch,),
        in_specs=[pl.BlockSpec(memory_space=pl.ANY)],
        out_specs=pl.BlockSpec(memory_space=pl.ANY),
        scratch_shapes=[
            pltpu.VMEM((2, n, n), jnp.float32),
            pltpu.SemaphoreType.DMA((2,)),
            pltpu.SemaphoreType.DMA((2,)),
        ],
        compiler_params=pltpu.CompilerParams(
            dimension_semantics=("arbitrary",),
            vmem_limit_bytes=48 * 1024 * 1024,
        ),
        name="sinkhorn_prob_domain",
    )(log_alpha)


# speedup vs baseline: 1.5375x; 1.5375x over previous
import jax
import jax.numpy as jnp
from jax.experimental import pallas as pl
from jax.experimental.pallas import tpu as pltpu

_N_ITERS = 20
_STRIP = 1024


_UNROLL = 1


def _normalize_in_place(e_ref, n):
    ns = n // _STRIP

    def sweep0(s, c_acc):
        rows = pl.ds(s * _STRIP, _STRIP)
        z = e_ref[rows, :]
        m = jnp.max(z, axis=1, keepdims=True)
        e = jnp.exp(z - m)
        e_ref[rows, :] = e
        r = jnp.sum(e, axis=1, keepdims=True)
        a = 1.0 / r
        ae = e_ref[rows, :] * a
        return c_acc + jnp.sum(ae, axis=0, keepdims=True)

    c = jax.lax.fori_loop(0, ns, sweep0, jnp.zeros((1, n), jnp.float32))
    b = 1.0 / c

    def iter_body(_, b):
        def sweep(s, c_acc):
            rows = pl.ds(s * _STRIP, _STRIP)
            f = e_ref[rows, :] * b
            r = jnp.sum(f, axis=1, keepdims=True)
            a = 1.0 / r
            ae = e_ref[rows, :] * a
            return c_acc + jnp.sum(ae, axis=0, keepdims=True)

        c = jax.lax.fori_loop(0, ns, sweep, jnp.zeros((1, n), jnp.float32))
        return 1.0 / c

    b_prev = jax.lax.fori_loop(0, _N_ITERS - 2, iter_body, b)
    b_last = iter_body(0, b_prev)

    w = b_last / b_prev

    def final_sweep(s, carry):
        rows = pl.ds(s * _STRIP, _STRIP)
        f = e_ref[rows, :] * b_prev
        r = jnp.sum(f, axis=1, keepdims=True)
        a = 1.0 / r
        e_ref[rows, :] = (f * a) * w
        return carry

    jax.lax.fori_loop(0, ns, final_sweep, 0)


def _sinkhorn_body(x_hbm, o_hbm, e_scr, in_sems, out_sems):
    i = pl.program_id(0)
    nb = pl.num_programs(0)
    n = e_scr.shape[1]
    buf = jax.lax.rem(i, 2)
    other = 1 - buf

    @pl.when(i == 0)
    def _():
        pltpu.make_async_copy(x_hbm.at[0], e_scr.at[0], in_sems.at[0]).start()
        pltpu.make_async_copy(x_hbm.at[1], e_scr.at[1], in_sems.at[1]).start()

    pltpu.make_async_copy(x_hbm.at[i], e_scr.at[buf], in_sems.at[buf]).wait()

    _normalize_in_place(e_scr.at[buf], n)

    pltpu.make_async_copy(e_scr.at[buf], o_hbm.at[i], out_sems.at[buf]).start()

    @pl.when(jnp.logical_and(i >= 1, i < nb - 1))
    def _():
        pltpu.make_async_copy(e_scr.at[other], o_hbm.at[i - 1],
                              out_sems.at[other]).wait()
        pltpu.make_async_copy(x_hbm.at[i + 1], e_scr.at[other],
                              in_sems.at[other]).start()

    @pl.when(i == nb - 1)
    def _():
        pltpu.make_async_copy(e_scr.at[other], o_hbm.at[i - 1],
                              out_sems.at[other]).wait()
        pltpu.make_async_copy(e_scr.at[buf], o_hbm.at[i],
                              out_sems.at[buf]).wait()


def kernel(log_alpha):
    batch, n, _ = log_alpha.shape
    return pl.pallas_call(
        _sinkhorn_body,
        out_shape=jax.ShapeDtypeStruct((batch, n, n), jnp.float32),
        grid=(batch,),
        in_specs=[pl.BlockSpec(memory_space=pl.ANY)],
        out_specs=pl.BlockSpec(memory_space=pl.ANY),
        scratch_shapes=[
            pltpu.VMEM((2, n, n), jnp.float32),
            pltpu.SemaphoreType.DMA((2,)),
            pltpu.SemaphoreType.DMA((2,)),
        ],
        compiler_params=pltpu.CompilerParams(
            dimension_semantics=("arbitrary",),
            vmem_limit_bytes=48 * 1024 * 1024,
        ),
        name="sinkhorn_prob_domain",
    )(log_alpha)
